# Initial kernel scaffold; baseline (speedup 1.0000x reference)
import jax
import jax.numpy as jnp
from jax.experimental import pallas as pl


def kernel(x, edge_index, edge_attr, Wl1, Wr1, We1, att1, b1, bn1_g, bn1_b, Wl2, Wr2, We2, att2, b2, bn2_g, bn2_b, Wp, bp, Wih, Whh, bih, bhh, Va, ba, Ua, Wf1, bf1, Wf2, bf2):
    raise NotImplementedError("write your pallas kernel here")


# speedup vs baseline: 2.3456x; 2.3456x over previous
import functools

import jax
import jax.numpy as jnp
from jax.experimental import pallas as pl
from jax.experimental.pallas import tpu as pltpu

def _elu(v):
    return jnp.where(v > 0, v, jnp.exp(jnp.minimum(v, 0.0)) - 1.0)


H, C = 2, 80
N_NODES = 23
NP = 24
E_EDGES = 506
EP = 512
FDIM = 160
HID = 192
F_IN = 6


def _gat_body(x_ref, S_ref, D_ref, Dt_ref, logD_ref, ea_ref, We_ref,
              Wl_ref, Wr_ref, attm_ref, b_ref, ss_ref, h_ref, stats_ref,
              *, G, fin, norm_in):
    i = pl.program_id(0)
    xt = x_ref[...]
    if norm_in:
        sc = ss_ref[0]
        sh = ss_ref[1]
        xt = _elu(xt * sc[None, None, :] + sh[None, None, :])
    x2 = xt.reshape(G * NP, fin)
    xl = jnp.dot(x2, Wl_ref[...], preferred_element_type=jnp.float32)
    xr = jnp.dot(x2, Wr_ref[...], preferred_element_type=jnp.float32)
    xl = xl.reshape(G, NP, FDIM)
    xr = xr.reshape(G, NP, FDIM)
    ef = jnp.dot(ea_ref[...], We_ref[...], preferred_element_type=jnp.float32)
    S = S_ref[...]
    D = D_ref[...]
    Dt = Dt_ref[...]
    attm = attm_ref[...]

    vsrcs = []
    logits = [[], []]
    for g in range(G):
        vs = jnp.dot(S, xl[g], preferred_element_type=jnp.float32)
        vd = jnp.dot(D, xr[g], preferred_element_type=jnp.float32)
        m = vs + vd + ef
        m = jnp.where(m >= 0, m, 0.2 * m)
        vsrcs.append(vs)
        for h in range(H):
            logits[h].append(jnp.sum(m * attm[h][None, :], axis=-1))

    alphas = []
    for h in range(H):
        lg = jnp.stack(logits[h])
        cols = []
        for n in range(NP):
            cols.append(jnp.max(lg + logD_ref[n][None, :], axis=-1, keepdims=True))
        lmax = jnp.concatenate(cols, axis=-1)
        lmax = jnp.where(jnp.isfinite(lmax), lmax, 0.0)
        lme = jnp.dot(lmax, Dt, preferred_element_type=jnp.float32)
        ex = jnp.exp(lg - lme)
        den = jnp.dot(ex, D, preferred_element_type=jnp.float32)
        dene = jnp.dot(den, Dt, preferred_element_type=jnp.float32)
        alphas.append(ex / (dene + 1e-16))

    b = b_ref[...]
    houts = []
    for g in range(G):
        per_head = []
        for h in range(H):
            w = vsrcs[g][:, h * C:(h + 1) * C] * alphas[h][g][:, None]
            per_head.append(jnp.dot(Dt, w, preferred_element_type=jnp.float32))
        houts.append(jnp.concatenate(per_head, axis=-1) + b[None, :])
    hout = jnp.stack(houts)
    h_ref[...] = hout

    real = hout[:, :N_NODES, :]
    s1 = jnp.sum(real, axis=(0, 1))
    s2 = jnp.sum(real * real, axis=(0, 1))
    upd = jnp.concatenate(
        [s1[None], s2[None], jnp.zeros((6, FDIM), jnp.float32)], axis=0)

    @pl.when(i == 0)
    def _():
        stats_ref[...] = jnp.zeros_like(stats_ref)

    stats_ref[...] = stats_ref[...] + upd


def _pool_body(h_ref, ss_ref, Wp_ref, bp_ref, o_ref, *, G):
    xt = _elu(h_ref[...] * ss_ref[0][None, None, :] + ss_ref[1][None, None, :])
    pooled = jnp.sum(xt[:, :N_NODES, :], axis=1) * (1.0 / N_NODES)
    o_ref[...] = jnp.dot(pooled, Wp_ref[...], preferred_element_type=jnp.float32) + bp_ref[...][None, :]


def _temporal_body(seq_ref, Wir_ref, Wiz_ref, Win_ref, bir_ref, biz_ref, bin_ref,
                   Whr_ref, Whz_ref, Whn_ref, bhr_ref, bhz_ref, bhn_ref,
                   Va_ref, ba_ref, Uat_ref, Wf1_ref, bf1_ref, Wf2_ref, bf2_ref,
                   o_ref, gir_s, giz_s, gin_s, outs_s, *, K, B):
    seq2 = seq_ref[...].reshape(K * B, HID)
    gir_s[...] = (jnp.dot(seq2, Wir_ref[...], preferred_element_type=jnp.float32)
                  + bir_ref[...][None, :]).reshape(K, B, HID)
    giz_s[...] = (jnp.dot(seq2, Wiz_ref[...], preferred_element_type=jnp.float32)
                  + biz_ref[...][None, :]).reshape(K, B, HID)
    gin_s[...] = (jnp.dot(seq2, Win_ref[...], preferred_element_type=jnp.float32)
                  + bin_ref[...][None, :]).reshape(K, B, HID)

    def step(k, hprev):
        ghr = jnp.dot(hprev, Whr_ref[...], preferred_element_type=jnp.float32) + bhr_ref[...][None, :]
        ghz = jnp.dot(hprev, Whz_ref[...], preferred_element_type=jnp.float32) + bhz_ref[...][None, :]
        ghn = jnp.dot(hprev, Whn_ref[...], preferred_element_type=jnp.float32) + bhn_ref[...][None, :]
        r = jax.nn.sigmoid(gir_s[k] + ghr)
        z = jax.nn.sigmoid(giz_s[k] + ghz)
        n = jnp.tanh(gin_s[k] + r * ghn)
        hnew = (1.0 - z) * n + z * hprev
        outs_s[k] = hnew
        return hnew

    jax.lax.fori_loop(0, K, step, jnp.zeros((B, HID), jnp.float32))

    outs = outs_s[...]
    t = jnp.tanh(jnp.dot(outs.reshape(K * B, HID), Va_ref[...],
                         preferred_element_type=jnp.float32) + ba_ref[...][None, :])
    score = jnp.sum(t * Uat_ref[...], axis=-1, keepdims=True).reshape(K, B, 1)
    mx = jnp.max(score, axis=0, keepdims=True)
    ex = jnp.exp(score - mx)
    al = ex / jnp.sum(ex, axis=0, keepdims=True)
    ctx = jnp.sum(al * outs, axis=0)
    f1 = jnp.maximum(jnp.dot(ctx, Wf1_ref[...], preferred_element_type=jnp.float32)
                     + bf1_ref[...][None, :], 0.0)
    o_ref[...] = jnp.dot(f1, Wf2_ref[...], preferred_element_type=jnp.float32) + bf2_ref[...][None, :]


def _full(shape):
    return pl.BlockSpec(shape, lambda i: tuple(0 for _ in shape))


def _attm(att):
    z = jnp.zeros((1, C), jnp.float32)
    return jnp.concatenate([
        jnp.concatenate([att[0:1], z], axis=1),
        jnp.concatenate([z, att[1:2]], axis=1)], axis=0)


def _bn_ss(stats, g, b, count):
    mu = stats[0] / count
    var = stats[1] / count - mu * mu
    rs = g * jax.lax.rsqrt(var + 1e-5)
    return jnp.concatenate([rs[None], (b - mu * rs)[None],
                            jnp.zeros((6, FDIM), jnp.float32)], axis=0)


def kernel(x, edge_index, edge_attr, Wl1, Wr1, We1, att1, b1, bn1_g, bn1_b,
           Wl2, Wr2, We2, att2, b2, bn2_g, bn2_b, Wp, bp,
           Wih, Whh, bih, bhh, Va, ba, Ua, Wf1, bf1, Wf2, bf2):
    B, K, N, F = x.shape
    BK = B * K
    G = 16
    G3 = 256

    x4 = jnp.pad(x.reshape(BK, N, F), ((0, 0), (0, NP - N), (0, 0)))
    iota = jnp.arange(NP, dtype=edge_index.dtype)
    S = jnp.pad((edge_index[0][:, None] == iota[None, :]).astype(jnp.float32),
                ((0, EP - E_EDGES), (0, 0)))
    D = jnp.pad((edge_index[1][:, None] == iota[None, :]).astype(jnp.float32),
                ((0, EP - E_EDGES), (0, 0)))
    Dt = D.T
    logD = jnp.where(Dt > 0, 0.0, -jnp.inf)
    ea = jnp.pad(edge_attr, ((0, EP - E_EDGES), (0, 0)))
    zss = jnp.zeros((8, FDIM), jnp.float32)

    gat_specs = [
        None,
        _full((EP, NP)), _full((EP, NP)), _full((NP, EP)), _full((NP, EP)),
        _full((EP, 2)), _full((2, FDIM)),
        None,
        None,
        _full((2, FDIM)), _full((FDIM,)), _full((8, FDIM)),
    ]

    def gat_call(xin, fin, norm_in, We, Wl, Wr, attm, b, ss):
        specs = list(gat_specs)
        specs[0] = pl.BlockSpec((G, NP, fin), lambda i: (i, 0, 0))
        specs[7] = _full((fin, FDIM))
        specs[8] = _full((fin, FDIM))
        return pl.pallas_call(
            functools.partial(_gat_body, G=G, fin=fin, norm_in=norm_in),
            grid=(BK // G,),
            in_specs=specs,
            out_specs=(pl.BlockSpec((G, NP, FDIM), lambda i: (i, 0, 0)),
                       pl.BlockSpec((8, FDIM), lambda i: (0, 0))),
            out_shape=(jax.ShapeDtypeStruct((BK, NP, FDIM), jnp.float32),
                       jax.ShapeDtypeStruct((8, FDIM), jnp.float32)),
        )(xin, S, D, Dt, logD, ea, We, Wl, Wr, attm, b, ss)

    h1, st1 = gat_call(x4, F, False, We1, Wl1, Wr1, _attm(att1), b1, zss)
    ss1 = _bn_ss(st1, bn1_g, bn1_b, float(BK * N_NODES))
    h2, st2 = gat_call(h1, FDIM, True, We2, Wl2, Wr2, _attm(att2), b2, ss1)
    ss2 = _bn_ss(st2, bn2_g, bn2_b, float(BK * N_NODES))

    seq = pl.pallas_call(
        functools.partial(_pool_body, G=G3),
        grid=(BK // G3,),
        in_specs=[pl.BlockSpec((G3, NP, FDIM), lambda i: (i, 0, 0)),
                  _full((8, FDIM)), _full((FDIM, HID)), _full((HID,))],
        out_specs=pl.BlockSpec((G3, HID), lambda i: (i, 0)),
        out_shape=jax.ShapeDtypeStruct((BK, HID), jnp.float32),
    )(h2, ss2, Wp, bp)

    seq_t = seq.reshape(B, K, HID).transpose(1, 0, 2)

    Wir, Wiz, Win = Wih[:, :HID], Wih[:, HID:2 * HID], Wih[:, 2 * HID:]
    Whr, Whz, Whn = Whh[:, :HID], Whh[:, HID:2 * HID], Whh[:, 2 * HID:]
    bir, biz, bin_ = bih[:HID], bih[HID:2 * HID], bih[2 * HID:]
    bhr, bhz, bhn = bhh[:HID], bhh[HID:2 * HID], bhh[2 * HID:]
    Uat = Ua.reshape(1, HID)

    y = pl.pallas_call(
        functools.partial(_temporal_body, K=K, B=B),
        grid=(1,),
        in_specs=[_full((K, B, HID))] + [_full((HID, HID))] * 3
                 + [_full((HID,))] * 3 + [_full((HID, HID))] * 3
                 + [_full((HID,))] * 3
                 + [_full((HID, HID)), _full((HID,)), _full((1, HID)),
                    _full((HID, 256)), _full((256,)), _full((256, 2)), _full((2,))],
        out_specs=_full((B, 2)),
        out_shape=jax.ShapeDtypeStruct((B, 2), jnp.float32),
        scratch_shapes=[pltpu.VMEM((K, B, HID), jnp.float32)] * 4,
    )(seq_t, Wir, Wiz, Win, bir, biz, bin_, Whr, Whz, Whn, bhr, bhz, bhn,
      Va, ba, Uat, Wf1, bf1, Wf2, bf2)
    return y


# speedup vs baseline: 3.4016x; 1.4502x over previous
import functools

import jax
import jax.numpy as jnp
from jax.experimental import pallas as pl
from jax.experimental.pallas import tpu as pltpu

def _elu(v):
    return jnp.where(v > 0, v, jnp.exp(jnp.minimum(v, 0.0)) - 1.0)


H, C = 2, 80
CP = 128
N_NODES = 23
NP = 24
E_EDGES = 506
EP = 512
FDIM = 2 * CP
HID = 192
F_IN = 6


def _gat_body(x_ref, S_ref, D_ref, Dt_ref, logD_ref, ea_ref, We_ref,
              Wl_ref, Wr_ref, attm_ref, b_ref, ss_ref, h_ref, stats_ref,
              *, G, fin, norm_in):
    i = pl.program_id(0)
    xt = x_ref[...]
    if norm_in:
        sc = ss_ref[0]
        sh = ss_ref[1]
        xt = _elu(xt * sc[None, None, :] + sh[None, None, :])
    x2 = xt.reshape(G * NP, fin)
    xl = jnp.dot(x2, Wl_ref[...], preferred_element_type=jnp.float32)
    xr = jnp.dot(x2, Wr_ref[...], preferred_element_type=jnp.float32)
    xl = xl.reshape(G, NP, FDIM)
    xr = xr.reshape(G, NP, FDIM)
    ef = jnp.dot(ea_ref[...], We_ref[...], preferred_element_type=jnp.float32)
    S = S_ref[...]
    D = D_ref[...]
    Dt = Dt_ref[...]
    attm = attm_ref[...]

    vsrcs = []
    logits = [[], []]
    for g in range(G):
        vs = jnp.dot(S, xl[g], preferred_element_type=jnp.float32)
        vd = jnp.dot(D, xr[g], preferred_element_type=jnp.float32)
        m = vs + vd + ef
        m = jnp.where(m >= 0, m, 0.2 * m)
        vsrcs.append(vs)
        for h in range(H):
            logits[h].append(jnp.sum(m * attm[h][None, :], axis=-1))

    alphas = []
    for h in range(H):
        lg = jnp.stack(logits[h])
        cols = []
        for n in range(NP):
            cols.append(jnp.max(lg + logD_ref[n][None, :], axis=-1, keepdims=True))
        lmax = jnp.concatenate(cols, axis=-1)
        lmax = jnp.where(jnp.isfinite(lmax), lmax, 0.0)
        lme = jnp.dot(lmax, Dt, preferred_element_type=jnp.float32)
        ex = jnp.exp(lg - lme)
        den = jnp.dot(ex, D, preferred_element_type=jnp.float32)
        dene = jnp.dot(den, Dt, preferred_element_type=jnp.float32)
        alphas.append(ex / (dene + 1e-16))

    b = b_ref[...]
    houts = []
    for g in range(G):
        per_head = []
        for h in range(H):
            w = vsrcs[g][:, h * CP:(h + 1) * CP] * alphas[h][g][:, None]
            per_head.append(jnp.dot(Dt, w, preferred_element_type=jnp.float32))
        houts.append(jnp.concatenate(per_head, axis=-1) + b[None, :])
    hout = jnp.stack(houts)
    h_ref[...] = hout

    real = hout[:, :N_NODES, :]
    s1 = jnp.sum(real, axis=(0, 1))
    s2 = jnp.sum(real * real, axis=(0, 1))
    upd = jnp.concatenate(
        [s1[None], s2[None], jnp.zeros((6, FDIM), jnp.float32)], axis=0)

    @pl.when(i == 0)
    def _():
        stats_ref[...] = jnp.zeros_like(stats_ref)

    stats_ref[...] = stats_ref[...] + upd


def _pool_body(h_ref, ss_ref, Wp_ref, bp_ref, o_ref, *, G):
    xt = _elu(h_ref[...] * ss_ref[0][None, None, :] + ss_ref[1][None, None, :])
    pooled = jnp.sum(xt[:, :N_NODES, :], axis=1) * (1.0 / N_NODES)
    o_ref[...] = jnp.dot(pooled, Wp_ref[...], preferred_element_type=jnp.float32) + bp_ref[...][None, :]


def _temporal_body(seq_ref, Wir_ref, Wiz_ref, Win_ref, bir_ref, biz_ref, bin_ref,
                   Whr_ref, Whz_ref, Whn_ref, bhr_ref, bhz_ref, bhn_ref,
                   Va_ref, ba_ref, Uat_ref, Wf1_ref, bf1_ref, Wf2_ref, bf2_ref,
                   o_ref, gir_s, giz_s, gin_s, outs_s, *, K, B):
    seq2 = seq_ref[...].reshape(K * B, HID)
    gir_s[...] = (jnp.dot(seq2, Wir_ref[...], preferred_element_type=jnp.float32)
                  + bir_ref[...][None, :]).reshape(K, B, HID)
    giz_s[...] = (jnp.dot(seq2, Wiz_ref[...], preferred_element_type=jnp.float32)
                  + biz_ref[...][None, :]).reshape(K, B, HID)
    gin_s[...] = (jnp.dot(seq2, Win_ref[...], preferred_element_type=jnp.float32)
                  + bin_ref[...][None, :]).reshape(K, B, HID)

    def step(k, hprev):
        ghr = jnp.dot(hprev, Whr_ref[...], preferred_element_type=jnp.float32) + bhr_ref[...][None, :]
        ghz = jnp.dot(hprev, Whz_ref[...], preferred_element_type=jnp.float32) + bhz_ref[...][None, :]
        ghn = jnp.dot(hprev, Whn_ref[...], preferred_element_type=jnp.float32) + bhn_ref[...][None, :]
        r = jax.nn.sigmoid(gir_s[k] + ghr)
        z = jax.nn.sigmoid(giz_s[k] + ghz)
        n = jnp.tanh(gin_s[k] + r * ghn)
        hnew = (1.0 - z) * n + z * hprev
        outs_s[k] = hnew
        return hnew

    jax.lax.fori_loop(0, K, step, jnp.zeros((B, HID), jnp.float32))

    outs = outs_s[...]
    t = jnp.tanh(jnp.dot(outs.reshape(K * B, HID), Va_ref[...],
                         preferred_element_type=jnp.float32) + ba_ref[...][None, :])
    score = jnp.sum(t * Uat_ref[...], axis=-1, keepdims=True).reshape(K, B, 1)
    mx = jnp.max(score, axis=0, keepdims=True)
    ex = jnp.exp(score - mx)
    al = ex / jnp.sum(ex, axis=0, keepdims=True)
    ctx = jnp.sum(al * outs, axis=0)
    f1 = jnp.maximum(jnp.dot(ctx, Wf1_ref[...], preferred_element_type=jnp.float32)
                     + bf1_ref[...][None, :], 0.0)
    o_ref[...] = jnp.dot(f1, Wf2_ref[...], preferred_element_type=jnp.float32) + bf2_ref[...][None, :]


def _full(shape):
    return pl.BlockSpec(shape, lambda i: tuple(0 for _ in shape))


def _attm(att):
    return (jnp.zeros((2, FDIM), jnp.float32)
            .at[0, 0:C].set(att[0]).at[1, CP:CP + C].set(att[1]))


_CIDX = (jnp.arange(2 * C) // C) * CP + (jnp.arange(2 * C) % C)


def _padcols(W):
    return jnp.zeros((W.shape[0], FDIM), W.dtype).at[:, _CIDX].set(W)


def _padrows(W):
    return jnp.zeros((FDIM, W.shape[1]), W.dtype).at[_CIDX, :].set(W)


def _padvec(v):
    return jnp.zeros((FDIM,), v.dtype).at[_CIDX].set(v)


def _bn_ss(stats, g, b, count):
    mu = stats[0] / count
    var = stats[1] / count - mu * mu
    rs = g * jax.lax.rsqrt(var + 1e-5)
    return jnp.concatenate([rs[None], (b - mu * rs)[None],
                            jnp.zeros((6, FDIM), jnp.float32)], axis=0)


def kernel(x, edge_index, edge_attr, Wl1, Wr1, We1, att1, b1, bn1_g, bn1_b,
           Wl2, Wr2, We2, att2, b2, bn2_g, bn2_b, Wp, bp,
           Wih, Whh, bih, bhh, Va, ba, Ua, Wf1, bf1, Wf2, bf2):
    B, K, N, F = x.shape
    BK = B * K
    G = 16
    G3 = 256

    x4 = jnp.pad(x.reshape(BK, N, F), ((0, 0), (0, NP - N), (0, 0)))
    iota = jnp.arange(NP, dtype=edge_index.dtype)
    S = jnp.pad((edge_index[0][:, None] == iota[None, :]).astype(jnp.float32),
                ((0, EP - E_EDGES), (0, 0)))
    D = jnp.pad((edge_index[1][:, None] == iota[None, :]).astype(jnp.float32),
                ((0, EP - E_EDGES), (0, 0)))
    Dt = D.T
    logD = jnp.where(Dt > 0, 0.0, -jnp.inf)
    ea = jnp.pad(edge_attr, ((0, EP - E_EDGES), (0, 0)))
    zss = jnp.zeros((8, FDIM), jnp.float32)

    gat_specs = [
        None,
        _full((EP, NP)), _full((EP, NP)), _full((NP, EP)), _full((NP, EP)),
        _full((EP, 2)), _full((2, FDIM)),
        None,
        None,
        _full((2, FDIM)), _full((FDIM,)), _full((8, FDIM)),
    ]

    def gat_call(xin, fin, norm_in, We, Wl, Wr, attm, b, ss):
        specs = list(gat_specs)
        specs[0] = pl.BlockSpec((G, NP, fin), lambda i: (i, 0, 0))
        specs[7] = _full((fin, FDIM))
        specs[8] = _full((fin, FDIM))
        return pl.pallas_call(
            functools.partial(_gat_body, G=G, fin=fin, norm_in=norm_in),
            grid=(BK // G,),
            in_specs=specs,
            out_specs=(pl.BlockSpec((G, NP, FDIM), lambda i: (i, 0, 0)),
                       pl.BlockSpec((8, FDIM), lambda i: (0, 0))),
            out_shape=(jax.ShapeDtypeStruct((BK, NP, FDIM), jnp.float32),
                       jax.ShapeDtypeStruct((8, FDIM), jnp.float32)),
        )(xin, S, D, Dt, logD, ea, We, Wl, Wr, attm, b, ss)

    h1, st1 = gat_call(x4, F, False, _padcols(We1), _padcols(Wl1), _padcols(Wr1),
                       _attm(att1), _padvec(b1), zss)
    ss1 = _bn_ss(st1, _padvec(bn1_g), _padvec(bn1_b), float(BK * N_NODES))
    h2, st2 = gat_call(h1, FDIM, True, _padcols(We2), _padrows(_padcols(Wl2)),
                       _padrows(_padcols(Wr2)), _attm(att2), _padvec(b2), ss1)
    ss2 = _bn_ss(st2, _padvec(bn2_g), _padvec(bn2_b), float(BK * N_NODES))

    seq = pl.pallas_call(
        functools.partial(_pool_body, G=G3),
        grid=(BK // G3,),
        in_specs=[pl.BlockSpec((G3, NP, FDIM), lambda i: (i, 0, 0)),
                  _full((8, FDIM)), _full((FDIM, HID)), _full((HID,))],
        out_specs=pl.BlockSpec((G3, HID), lambda i: (i, 0)),
        out_shape=jax.ShapeDtypeStruct((BK, HID), jnp.float32),
    )(h2, ss2, _padrows(Wp), bp)

    seq_t = seq.reshape(B, K, HID).transpose(1, 0, 2)

    Wir, Wiz, Win = Wih[:, :HID], Wih[:, HID:2 * HID], Wih[:, 2 * HID:]
    Whr, Whz, Whn = Whh[:, :HID], Whh[:, HID:2 * HID], Whh[:, 2 * HID:]
    bir, biz, bin_ = bih[:HID], bih[HID:2 * HID], bih[2 * HID:]
    bhr, bhz, bhn = bhh[:HID], bhh[HID:2 * HID], bhh[2 * HID:]
    Uat = Ua.reshape(1, HID)

    y = pl.pallas_call(
        functools.partial(_temporal_body, K=K, B=B),
        grid=(1,),
        in_specs=[_full((K, B, HID))] + [_full((HID, HID))] * 3
                 + [_full((HID,))] * 3 + [_full((HID, HID))] * 3
                 + [_full((HID,))] * 3
                 + [_full((HID, HID)), _full((HID,)), _full((1, HID)),
                    _full((HID, 256)), _full((256,)), _full((256, 2)), _full((2,))],
        out_specs=_full((B, 2)),
        out_shape=jax.ShapeDtypeStruct((B, 2), jnp.float32),
        scratch_shapes=[pltpu.VMEM((K, B, HID), jnp.float32)] * 4,
    )(seq_t, Wir, Wiz, Win, bir, biz, bin_, Whr, Whz, Whn, bhr, bhz, bhn,
      Va, ba, Uat, Wf1, bf1, Wf2, bf2)
    return y
